# baseline (speedup 1.0000x reference)
import functools

import jax
import jax.numpy as jnp
from jax import lax
from jax.experimental import pallas as pl
from jax.experimental.pallas import tpu as pltpu
from jax.experimental.pallas import tpu_sc as plsc

B, N = 2, 1024
DIM, HIDDEN = 128, 4
KNN, M_DIM = 32, 16
EDGE_IN = 2 * DIM + 1
ROWS = 256
EROWS = 128
GW = DIM


def _silu(t):
    return t * (1.0 / (1.0 + jnp.exp(-t)))


def _ln(x, g, b):
    mu = jnp.mean(x, axis=-1, keepdims=True)
    var = jnp.mean((x - mu) ** 2, axis=-1, keepdims=True)
    return (x - mu) / jnp.sqrt(var + 1e-5) * g + b


def _dot(a, b):
    return jnp.dot(a, b, preferred_element_type=jnp.float32)


def _embed_body(z_ref, tok_ref, pos_ref, out_ref):
    z = z_ref[0]
    num_tok = tok_ref.shape[0]
    cols = lax.broadcasted_iota(jnp.int32, (N, num_tok), 1)
    onehot = (z == cols).astype(jnp.float32)
    emb = jnp.dot(onehot, tok_ref[...], precision=lax.Precision.HIGHEST,
                  preferred_element_type=jnp.float32)
    out_ref[0] = emb + pos_ref[...]


def _embed(z, token_emb, pos_emb):
    z2 = z.reshape(B, N, 1).astype(jnp.int32)
    return pl.pallas_call(
        _embed_body,
        grid=(B,),
        in_specs=[
            pl.BlockSpec((1, N, 1), lambda b: (b, 0, 0)),
            pl.BlockSpec(token_emb.shape, lambda b: (0, 0)),
            pl.BlockSpec(pos_emb.shape, lambda b: (0, 0)),
        ],
        out_specs=pl.BlockSpec((1, N, DIM), lambda b: (b, 0, 0)),
        out_shape=jax.ShapeDtypeStruct((B, N, DIM), jnp.float32),
    )(z2, token_emb, pos_emb)


def _topk_body(coorsT_ref, coors_all_ref, coors_blk_ref, idx_ref, dist_ref, rel_ref, d_ref):
    coors_blk = coors_blk_ref[0]

    d = None
    for c in range(3):
        xi_c = coors_blk[:, c:c + 1]
        xj_c = coorsT_ref[0, c:c + 1, :]
        rel_c = xi_c - xj_c
        sq = rel_c * rel_c
        d = sq if d is None else d + sq

    cols = lax.broadcasted_iota(jnp.int32, (ROWS, N), 1)
    d_ref[...] = (lax.bitcast_convert_type(d, jnp.int32) & ~(N - 1)) | cols

    def body(k, carry):
        dcur = d_ref[...]
        mkey = jnp.min(dcur, axis=1, keepdims=True)
        hot = dcur == mkey
        d_ref[...] = jnp.where(hot, jnp.iinfo(jnp.int32).max, dcur)
        idx_ref[0, k] = mkey & (N - 1)
        dist_ref[0, k] = lax.bitcast_convert_type(mkey & ~(N - 1), jnp.float32)
        xj = jnp.dot(hot.astype(jnp.float32), coors_all_ref[0],
                     precision=lax.Precision.HIGHEST,
                     preferred_element_type=jnp.float32)
        coors_blk = coors_blk_ref[0]
        for c in range(3):
            rel_ref[0, k, :, c:c + 1] = coors_blk[:, c:c + 1] - xj[:, c:c + 1]
        return carry

    lax.fori_loop(0, KNN, body, 0)


def _topk(coors):
    coorsT = jnp.swapaxes(coors, 1, 2)
    perb = lambda shape: pl.BlockSpec((1,) + shape, lambda b, r: (b,) + (0,) * len(shape))
    blk = lambda shape: pl.BlockSpec((1, ROWS) + shape, lambda b, r: (b, r) + (0,) * len(shape))
    kblk = lambda w: pl.BlockSpec((1, KNN, ROWS, w), lambda b, r: (b, 0, r, 0))
    return pl.pallas_call(
        _topk_body,
        grid=(B, N // ROWS),
        in_specs=[perb((3, N)), perb((N, 3)), blk((3,))],
        out_specs=[kblk(1), kblk(1), kblk(3)],
        out_shape=[
            jax.ShapeDtypeStruct((B, KNN, N, 1), jnp.int32),
            jax.ShapeDtypeStruct((B, KNN, N, 1), jnp.float32),
            jax.ShapeDtypeStruct((B, KNN, N, 3), jnp.float32),
        ],
        scratch_shapes=[pltpu.VMEM((ROWS, N), jnp.int32)],
    )(coorsT, coors, coors)


_NC, _NS = 2, 16
_NW = _NC * _NS
_E_TOTAL = B * KNN * N
_E_PER_W = _E_TOTAL // _NW
_CHUNK = 512


def _sc_gather_kernel(idx_hbm, table_hbm, out_hbm, idx_v, buf_v, sem):
    wid = lax.axis_index("s") * _NC + lax.axis_index("c")
    base = wid * _E_PER_W
    for ch in range(_E_PER_W // _CHUNK):
        off = base + ch * _CHUNK
        pltpu.sync_copy(idx_hbm.at[pl.ds(off, _CHUNK)], idx_v)
        pltpu.async_copy(table_hbm.at[idx_v], buf_v, sem).wait()
        pltpu.sync_copy(buf_v, out_hbm.at[pl.ds(off, _CHUNK)])


def _sc_gather(table, eidx):
    mesh = plsc.VectorSubcoreMesh(core_axis_name="c", subcore_axis_name="s")
    k = functools.partial(
        pl.kernel,
        mesh=mesh,
        out_type=jax.ShapeDtypeStruct((_E_TOTAL, GW), jnp.float32),
        scratch_types=[
            pltpu.VMEM((_CHUNK,), jnp.int32),
            pltpu.VMEM((_CHUNK, GW), jnp.float32),
            pltpu.SemaphoreType.DMA,
        ],
    )(_sc_gather_kernel)
    return k(eidx, table)


def _edge_body(
    g_ref, dist_ref, rel_ref, feats_blk_ref, coors_blk_ref,
    w1a_ref, w1b_ref, w1d_ref, b1_ref, w2_ref, b2_ref,
    cw1_ref, cb1_ref, cw2_ref, cb2_ref, cns_ref,
    nng_ref, nnb_ref, nw1a_ref, nw1b_ref, nb1_ref, nw2_ref, nb2_ref,
    feats_out_ref, coors_out_ref,
):
    feats_blk = feats_blk_ref[0]
    coors_blk = coors_blk_ref[0]
    ne = KNN * EROWS

    gi = _dot(feats_blk, w1a_ref[...]) + b1_ref[...]

    fj = g_ref[0].reshape(ne, DIM).astype(jnp.bfloat16)
    mval = dist_ref[0].reshape(ne, 1)
    pre = (_dot(fj, w1b_ref[...]) + mval * w1d_ref[...]).reshape(
        KNN, EROWS, 2 * EDGE_IN) + gi[None]
    h = _silu(pre).reshape(ne, 2 * EDGE_IN)
    mk = _silu(_dot(h.astype(jnp.bfloat16), w2_ref[...]) + b2_ref[...])
    cwh = _silu(_dot(mk, cw1_ref[...]) + cb1_ref[...])
    cw = _dot(cwh, cw2_ref[...]) + cb2_ref[...]
    scale = cw / jnp.clip(jnp.sqrt(mval), 1e-8) * cns_ref[0, 0]

    m_i = jnp.sum(mk.reshape(KNN, EROWS, M_DIM), axis=0)
    scale3 = scale.reshape(KNN, EROWS, 1)
    rel3 = rel_ref[0]
    cds = [jnp.sum(scale3[..., 0] * rel3[..., c], axis=0)[:, None]
           for c in range(3)]
    cdelta = jnp.concatenate(cds, axis=1)

    nf = _ln(feats_blk, nng_ref[...], nnb_ref[...])
    pre2 = _dot(nf, nw1a_ref[...]) + _dot(m_i, nw1b_ref[...]) + nb1_ref[...]
    node_out = _dot(_silu(pre2), nw2_ref[...]) + nb2_ref[...] + feats_blk

    feats_out_ref[0] = node_out
    coors_out_ref[0] = coors_blk + cdelta


def _layer(feats, coors, lp):
    nbr_idx, dist_t, rel_t = _topk(coors)
    boff = (jnp.arange(B, dtype=jnp.int32) * N)[:, None, None, None]
    eidx = (nbr_idx + boff).reshape(-1)
    g = _sc_gather(feats.reshape(B * N, GW), eidx).reshape(B, KNN, N, GW)

    w1a = lp['e_W1'][:DIM]
    w1b = lp['e_W1'][DIM:2 * DIM].astype(jnp.bfloat16)
    w2b = lp['e_W2'].astype(jnp.bfloat16)
    w1d = lp['e_W1'][2 * DIM:2 * DIM + 1]
    nw1a = lp['n_W1'][:DIM]
    nw1b = lp['n_W1'][DIM:]
    row2 = lambda v: v.reshape(1, -1)

    grid = (B, N // EROWS)
    full = lambda shape: pl.BlockSpec(shape, lambda b, r: (0,) * len(shape))
    blk = lambda shape: pl.BlockSpec((1, EROWS) + shape, lambda b, r: (b, r) + (0,) * len(shape))
    gblk = pl.BlockSpec((1, KNN, EROWS, GW), lambda b, r: (b, 0, r, 0))
    dblk = pl.BlockSpec((1, KNN, EROWS, 1), lambda b, r: (b, 0, r, 0))
    rblk = pl.BlockSpec((1, KNN, EROWS, 3), lambda b, r: (b, 0, r, 0))

    out = pl.pallas_call(
        _edge_body,
        grid=grid,
        in_specs=[
            gblk, dblk, rblk,
            blk((DIM,)),
            blk((3,)),
            full(w1a.shape), full(w1b.shape), full((1, 2 * EDGE_IN)),
            full((1, 2 * EDGE_IN)),
            full(w2b.shape), full((1, M_DIM)),
            full(lp['c_W1'].shape), full((1, 4 * M_DIM)),
            full(lp['c_W2'].shape), full((1, 1)), full((1, 1)),
            full((1, DIM)), full((1, DIM)),
            full(nw1a.shape), full(nw1b.shape), full((1, 2 * DIM)),
            full(lp['n_W2'].shape), full((1, DIM)),
        ],
        out_specs=[blk((DIM,)), blk((3,))],
        out_shape=[
            jax.ShapeDtypeStruct((B, N, DIM), jnp.float32),
            jax.ShapeDtypeStruct((B, N, 3), jnp.float32),
        ],
    )(
        g, dist_t, rel_t, feats, coors,
        w1a, w1b, row2(w1d), row2(lp['e_b1']), w2b, row2(lp['e_b2']),
        lp['c_W1'], row2(lp['c_b1']), lp['c_W2'], row2(lp['c_b2']),
        row2(lp['cn_scale']),
        row2(lp['nn_g']), row2(lp['nn_b']),
        nw1a, nw1b, row2(lp['n_b1']), lp['n_W2'], row2(lp['n_b2']),
    )
    return out[0], out[1]


def _head_body(x_ref, w1_ref, b1_ref, w2_ref, b2_ref, pa_ref,
               g1_ref, bb1_ref, g2_ref, bb2_ref, out_ref):
    x = x_ref[0]
    h = _ln(x + x, g1_ref[...], bb1_ref[...])
    a = _dot(h, w1_ref[...]) + b1_ref[...]
    a = jnp.where(a >= 0, a, pa_ref[0, 0] * a)
    h2 = _dot(a, w2_ref[...]) + b2_ref[...]
    out_ref[0] = _ln(h + h2, g2_ref[...], bb2_ref[...])


def _head(feats, params):
    row2 = lambda v: v.reshape(1, -1)
    full = lambda shape: pl.BlockSpec(shape, lambda b: (0,) * len(shape))
    return pl.pallas_call(
        _head_body,
        grid=(B,),
        in_specs=[
            pl.BlockSpec((1, N, DIM), lambda b: (b, 0, 0)),
            full(params['ffn_W1'].shape), full((1, HIDDEN * DIM)),
            full(params['ffn_W2'].shape), full((1, DIM)),
            full((1, 1)),
            full((1, DIM)), full((1, DIM)), full((1, DIM)), full((1, DIM)),
        ],
        out_specs=pl.BlockSpec((1, N, DIM), lambda b: (b, 0, 0)),
        out_shape=jax.ShapeDtypeStruct((B, N, DIM), jnp.float32),
    )(
        feats,
        params['ffn_W1'], row2(params['ffn_b1']),
        params['ffn_W2'], row2(params['ffn_b2']),
        row2(params['prelu_a']),
        row2(params['norm1_g']), row2(params['norm1_b']),
        row2(params['norm2_g']), row2(params['norm2_b']),
    )


@jax.jit
def kernel(x, z, params):
    feats = _embed(z, params['token_emb'], params['pos_emb'][:N])
    coors = x
    for lp in params['layers']:
        feats, coors = _layer(feats, coors, lp)
    h = _head(feats, params)
    return h, coors


# speedup vs baseline: 1.4546x; 1.4546x over previous
import functools

import jax
import jax.numpy as jnp
from jax import lax
from jax.experimental import pallas as pl
from jax.experimental.pallas import tpu as pltpu
from jax.experimental.pallas import tpu_sc as plsc

B, N = 2, 1024
DIM, HIDDEN = 128, 4
KNN, M_DIM = 32, 16
EDGE_IN = 2 * DIM + 1
ROWS = 256
EROWS = 128
GW = DIM


def _silu(t):
    return t * (1.0 / (1.0 + jnp.exp(-t)))


def _ln(x, g, b):
    mu = jnp.mean(x, axis=-1, keepdims=True)
    var = jnp.mean((x - mu) ** 2, axis=-1, keepdims=True)
    return (x - mu) / jnp.sqrt(var + 1e-5) * g + b


def _dot(a, b):
    return jnp.dot(a, b, preferred_element_type=jnp.float32)


def _embed_body(z_ref, tok_ref, pos_ref, out_ref):
    z = z_ref[0]
    num_tok = tok_ref.shape[0]
    cols = lax.broadcasted_iota(jnp.int32, (N, num_tok), 1)
    onehot = (z == cols).astype(jnp.float32)
    emb = jnp.dot(onehot, tok_ref[...], precision=lax.Precision.HIGHEST,
                  preferred_element_type=jnp.float32)
    out_ref[0] = emb + pos_ref[...]


def _embed(z, token_emb, pos_emb):
    z2 = z.reshape(B, N, 1).astype(jnp.int32)
    return pl.pallas_call(
        _embed_body,
        grid=(B,),
        in_specs=[
            pl.BlockSpec((1, N, 1), lambda b: (b, 0, 0)),
            pl.BlockSpec(token_emb.shape, lambda b: (0, 0)),
            pl.BlockSpec(pos_emb.shape, lambda b: (0, 0)),
        ],
        out_specs=pl.BlockSpec((1, N, DIM), lambda b: (b, 0, 0)),
        out_shape=jax.ShapeDtypeStruct((B, N, DIM), jnp.float32),
    )(z2, token_emb, pos_emb)


def _topk_body(coorsT_ref, coors_blk_ref, idx_ref, dist_ref, rel_ref, d_ref):
    coors_blk = coors_blk_ref[0]

    d = None
    for c in range(3):
        xi_c = coors_blk[:, c:c + 1]
        xj_c = coorsT_ref[0, c:c + 1, :]
        rel_c = xi_c - xj_c
        sq = rel_c * rel_c
        d = sq if d is None else d + sq

    cols = lax.broadcasted_iota(jnp.int32, (ROWS, N), 1)
    d_ref[...] = (lax.bitcast_convert_type(d, jnp.int32) & ~(N - 1)) | cols

    def body(k, carry):
        dcur = d_ref[...]
        mkey = jnp.min(dcur, axis=1, keepdims=True)
        hot = dcur == mkey
        d_ref[...] = jnp.where(hot, jnp.iinfo(jnp.int32).max, dcur)
        idx_ref[0, k] = mkey & (N - 1)
        dist_ref[0, k] = lax.bitcast_convert_type(mkey & ~(N - 1), jnp.float32)
        coors_blk = coors_blk_ref[0]
        for c in range(3):
            xj_c = jnp.sum(jnp.where(hot, coorsT_ref[0, c:c + 1, :], 0.0),
                           axis=1, keepdims=True)
            rel_ref[0, k, :, c:c + 1] = coors_blk[:, c:c + 1] - xj_c
        return carry

    lax.fori_loop(0, KNN, body, 0)


def _topk(coors):
    coorsT = jnp.swapaxes(coors, 1, 2)
    perb = lambda shape: pl.BlockSpec((1,) + shape, lambda b, r: (b,) + (0,) * len(shape))
    blk = lambda shape: pl.BlockSpec((1, ROWS) + shape, lambda b, r: (b, r) + (0,) * len(shape))
    kblk = lambda w: pl.BlockSpec((1, KNN, ROWS, w), lambda b, r: (b, 0, r, 0))
    return pl.pallas_call(
        _topk_body,
        grid=(B, N // ROWS),
        in_specs=[perb((3, N)), blk((3,))],
        out_specs=[kblk(1), kblk(1), kblk(3)],
        out_shape=[
            jax.ShapeDtypeStruct((B, KNN, N, 1), jnp.int32),
            jax.ShapeDtypeStruct((B, KNN, N, 1), jnp.float32),
            jax.ShapeDtypeStruct((B, KNN, N, 3), jnp.float32),
        ],
        scratch_shapes=[pltpu.VMEM((ROWS, N), jnp.int32)],
    )(coorsT, coors)


_NC, _NS = 2, 16
_NW = _NC * _NS
_E_TOTAL = B * KNN * N
_E_PER_W = _E_TOTAL // _NW
_CHUNK = 256


def _sc_gather_kernel(idx_hbm, table_hbm, out_hbm, idx_v, buf0, buf1, sem0, sem1):
    wid = lax.axis_index("s") * _NC + lax.axis_index("c")
    base = wid * _E_PER_W
    nch = _E_PER_W // _CHUNK
    pltpu.sync_copy(idx_hbm.at[pl.ds(base, _E_PER_W)], idx_v)
    bufs = (buf0, buf1)
    sems = (sem0, sem1)
    cps = []
    for ch in range(nch):
        cp = pltpu.async_copy(table_hbm.at[idx_v.at[pl.ds(ch * _CHUNK, _CHUNK)]],
                              bufs[ch % 2], sems[ch % 2])
        cps.append(cp)
        if ch >= 1:
            cps[ch - 1].wait()
            pltpu.sync_copy(bufs[(ch - 1) % 2],
                            out_hbm.at[pl.ds(base + (ch - 1) * _CHUNK, _CHUNK)])
    cps[nch - 1].wait()
    pltpu.sync_copy(bufs[(nch - 1) % 2],
                    out_hbm.at[pl.ds(base + (nch - 1) * _CHUNK, _CHUNK)])


def _sc_gather(table, eidx):
    mesh = plsc.VectorSubcoreMesh(core_axis_name="c", subcore_axis_name="s")
    k = functools.partial(
        pl.kernel,
        mesh=mesh,
        out_type=jax.ShapeDtypeStruct((_E_TOTAL, GW), jnp.float32),
        scratch_types=[
            pltpu.VMEM((_E_PER_W,), jnp.int32),
            pltpu.VMEM((_CHUNK, GW), jnp.float32),
            pltpu.VMEM((_CHUNK, GW), jnp.float32),
            pltpu.SemaphoreType.DMA,
            pltpu.SemaphoreType.DMA,
        ],
    )(_sc_gather_kernel)
    return k(eidx, table)


def _edge_body(
    g_ref, dist_ref, rel_ref, feats_blk_ref, coors_blk_ref,
    w1a_ref, w1b_ref, w1d_ref, b1_ref, w2_ref, b2_ref,
    cw1_ref, cb1_ref, cw2_ref, cb2_ref, cns_ref,
    nng_ref, nnb_ref, nw1a_ref, nw1b_ref, nb1_ref, nw2_ref, nb2_ref,
    feats_out_ref, coors_out_ref,
):
    feats_blk = feats_blk_ref[0]
    coors_blk = coors_blk_ref[0]
    ne = KNN * EROWS

    gi = _dot(feats_blk, w1a_ref[...]) + b1_ref[...]

    fj = g_ref[0].reshape(ne, DIM).astype(jnp.bfloat16)
    mval = dist_ref[0].reshape(ne, 1)
    pre = (_dot(fj, w1b_ref[...]) + mval * w1d_ref[...]).reshape(
        KNN, EROWS, 2 * EDGE_IN) + gi[None]
    h = _silu(pre).reshape(ne, 2 * EDGE_IN)
    mk = _silu(_dot(h.astype(jnp.bfloat16), w2_ref[...]) + b2_ref[...])
    cwh = _silu(_dot(mk, cw1_ref[...]) + cb1_ref[...])
    cw = _dot(cwh, cw2_ref[...]) + cb2_ref[...]
    scale = cw / jnp.clip(jnp.sqrt(mval), 1e-8) * cns_ref[0, 0]

    m_i = jnp.sum(mk.reshape(KNN, EROWS, M_DIM), axis=0)
    scale3 = scale.reshape(KNN, EROWS, 1)
    rel3 = rel_ref[0]
    cds = [jnp.sum(scale3[..., 0] * rel3[..., c], axis=0)[:, None]
           for c in range(3)]
    cdelta = jnp.concatenate(cds, axis=1)

    nf = _ln(feats_blk, nng_ref[...], nnb_ref[...])
    pre2 = _dot(nf, nw1a_ref[...]) + _dot(m_i, nw1b_ref[...]) + nb1_ref[...]
    node_out = _dot(_silu(pre2), nw2_ref[...]) + nb2_ref[...] + feats_blk

    feats_out_ref[0] = node_out
    coors_out_ref[0] = coors_blk + cdelta


def _layer(feats, coors, lp):
    nbr_idx, dist_t, rel_t = _topk(coors)
    boff = (jnp.arange(B, dtype=jnp.int32) * N)[:, None, None, None]
    eidx = (nbr_idx + boff).reshape(-1)
    g = _sc_gather(feats.reshape(B * N, GW), eidx).reshape(B, KNN, N, GW)

    w1a = lp['e_W1'][:DIM]
    w1b = lp['e_W1'][DIM:2 * DIM].astype(jnp.bfloat16)
    w2b = lp['e_W2'].astype(jnp.bfloat16)
    w1d = lp['e_W1'][2 * DIM:2 * DIM + 1]
    nw1a = lp['n_W1'][:DIM]
    nw1b = lp['n_W1'][DIM:]
    row2 = lambda v: v.reshape(1, -1)

    grid = (B, N // EROWS)
    full = lambda shape: pl.BlockSpec(shape, lambda b, r: (0,) * len(shape))
    blk = lambda shape: pl.BlockSpec((1, EROWS) + shape, lambda b, r: (b, r) + (0,) * len(shape))
    gblk = pl.BlockSpec((1, KNN, EROWS, GW), lambda b, r: (b, 0, r, 0))
    dblk = pl.BlockSpec((1, KNN, EROWS, 1), lambda b, r: (b, 0, r, 0))
    rblk = pl.BlockSpec((1, KNN, EROWS, 3), lambda b, r: (b, 0, r, 0))

    out = pl.pallas_call(
        _edge_body,
        grid=grid,
        in_specs=[
            gblk, dblk, rblk,
            blk((DIM,)),
            blk((3,)),
            full(w1a.shape), full(w1b.shape), full((1, 2 * EDGE_IN)),
            full((1, 2 * EDGE_IN)),
            full(w2b.shape), full((1, M_DIM)),
            full(lp['c_W1'].shape), full((1, 4 * M_DIM)),
            full(lp['c_W2'].shape), full((1, 1)), full((1, 1)),
            full((1, DIM)), full((1, DIM)),
            full(nw1a.shape), full(nw1b.shape), full((1, 2 * DIM)),
            full(lp['n_W2'].shape), full((1, DIM)),
        ],
        out_specs=[blk((DIM,)), blk((3,))],
        out_shape=[
            jax.ShapeDtypeStruct((B, N, DIM), jnp.float32),
            jax.ShapeDtypeStruct((B, N, 3), jnp.float32),
        ],
    )(
        g, dist_t, rel_t, feats, coors,
        w1a, w1b, row2(w1d), row2(lp['e_b1']), w2b, row2(lp['e_b2']),
        lp['c_W1'], row2(lp['c_b1']), lp['c_W2'], row2(lp['c_b2']),
        row2(lp['cn_scale']),
        row2(lp['nn_g']), row2(lp['nn_b']),
        nw1a, nw1b, row2(lp['n_b1']), lp['n_W2'], row2(lp['n_b2']),
    )
    return out[0], out[1]


def _head_body(x_ref, w1_ref, b1_ref, w2_ref, b2_ref, pa_ref,
               g1_ref, bb1_ref, g2_ref, bb2_ref, out_ref):
    x = x_ref[0]
    h = _ln(x + x, g1_ref[...], bb1_ref[...])
    a = _dot(h, w1_ref[...]) + b1_ref[...]
    a = jnp.where(a >= 0, a, pa_ref[0, 0] * a)
    h2 = _dot(a, w2_ref[...]) + b2_ref[...]
    out_ref[0] = _ln(h + h2, g2_ref[...], bb2_ref[...])


def _head(feats, params):
    row2 = lambda v: v.reshape(1, -1)
    full = lambda shape: pl.BlockSpec(shape, lambda b: (0,) * len(shape))
    return pl.pallas_call(
        _head_body,
        grid=(B,),
        in_specs=[
            pl.BlockSpec((1, N, DIM), lambda b: (b, 0, 0)),
            full(params['ffn_W1'].shape), full((1, HIDDEN * DIM)),
            full(params['ffn_W2'].shape), full((1, DIM)),
            full((1, 1)),
            full((1, DIM)), full((1, DIM)), full((1, DIM)), full((1, DIM)),
        ],
        out_specs=pl.BlockSpec((1, N, DIM), lambda b: (b, 0, 0)),
        out_shape=jax.ShapeDtypeStruct((B, N, DIM), jnp.float32),
    )(
        feats,
        params['ffn_W1'], row2(params['ffn_b1']),
        params['ffn_W2'], row2(params['ffn_b2']),
        row2(params['prelu_a']),
        row2(params['norm1_g']), row2(params['norm1_b']),
        row2(params['norm2_g']), row2(params['norm2_b']),
    )


@jax.jit
def kernel(x, z, params):
    feats = _embed(z, params['token_emb'], params['pos_emb'][:N])
    coors = x
    for lp in params['layers']:
        feats, coors = _layer(feats, coors, lp)
    h = _head(feats, params)
    return h, coors


# speedup vs baseline: 1.7654x; 1.2137x over previous
import functools

import jax
import jax.numpy as jnp
from jax import lax
from jax.experimental import pallas as pl
from jax.experimental.pallas import tpu as pltpu
from jax.experimental.pallas import tpu_sc as plsc

B, N = 2, 1024
DIM, HIDDEN = 128, 4
KNN, M_DIM = 32, 16
EDGE_IN = 2 * DIM + 1
ROWS = 256
EROWS = 128
GW = 2 * DIM


def _silu(t):
    return t * (1.0 / (1.0 + jnp.exp(-t)))


def _ln(x, g, b):
    mu = jnp.mean(x, axis=-1, keepdims=True)
    var = jnp.mean((x - mu) ** 2, axis=-1, keepdims=True)
    return (x - mu) / jnp.sqrt(var + 1e-5) * g + b


def _dot(a, b):
    return jnp.dot(a, b, preferred_element_type=jnp.float32)


def _embed_body(z_ref, tok_ref, pos_ref, out_ref):
    z = z_ref[0]
    num_tok = tok_ref.shape[0]
    cols = lax.broadcasted_iota(jnp.int32, (N, num_tok), 1)
    onehot = (z == cols).astype(jnp.float32)
    emb = jnp.dot(onehot, tok_ref[...], precision=lax.Precision.HIGHEST,
                  preferred_element_type=jnp.float32)
    out_ref[0] = emb + pos_ref[...]


def _embed(z, token_emb, pos_emb):
    z2 = z.reshape(B, N, 1).astype(jnp.int32)
    return pl.pallas_call(
        _embed_body,
        grid=(B,),
        in_specs=[
            pl.BlockSpec((1, N, 1), lambda b: (b, 0, 0)),
            pl.BlockSpec(token_emb.shape, lambda b: (0, 0)),
            pl.BlockSpec(pos_emb.shape, lambda b: (0, 0)),
        ],
        out_specs=pl.BlockSpec((1, N, DIM), lambda b: (b, 0, 0)),
        out_shape=jax.ShapeDtypeStruct((B, N, DIM), jnp.float32),
    )(z2, token_emb, pos_emb)


def _topk_body(coorsT_ref, coors_blk_ref, idx_ref, dist_ref, d_ref):
    coors_blk = coors_blk_ref[0]

    d = None
    for c in range(3):
        xi_c = coors_blk[:, c:c + 1]
        xj_c = coorsT_ref[0, c:c + 1, :]
        rel_c = xi_c - xj_c
        sq = rel_c * rel_c
        d = sq if d is None else d + sq

    cols = lax.broadcasted_iota(jnp.int32, (ROWS, N), 1)
    d_ref[...] = (lax.bitcast_convert_type(d, jnp.int32) & ~(N - 1)) | cols

    def body(k, carry):
        dcur = d_ref[...]
        mkey = jnp.min(dcur, axis=1, keepdims=True)
        hot = dcur == mkey
        d_ref[...] = jnp.where(hot, jnp.iinfo(jnp.int32).max, dcur)
        idx_ref[0, k] = mkey & (N - 1)
        dist_ref[0, k] = lax.bitcast_convert_type(mkey & ~(N - 1), jnp.float32)
        return carry

    lax.fori_loop(0, KNN, body, 0)


def _topk(coors):
    coorsT = jnp.swapaxes(coors, 1, 2)
    perb = lambda shape: pl.BlockSpec((1,) + shape, lambda b, r: (b,) + (0,) * len(shape))
    blk = lambda shape: pl.BlockSpec((1, ROWS) + shape, lambda b, r: (b, r) + (0,) * len(shape))
    kblk = lambda w: pl.BlockSpec((1, KNN, ROWS, w), lambda b, r: (b, 0, r, 0))
    return pl.pallas_call(
        _topk_body,
        grid=(B, N // ROWS),
        in_specs=[perb((3, N)), blk((3,))],
        out_specs=[kblk(1), kblk(1)],
        out_shape=[
            jax.ShapeDtypeStruct((B, KNN, N, 1), jnp.int32),
            jax.ShapeDtypeStruct((B, KNN, N, 1), jnp.float32),
        ],
        scratch_shapes=[pltpu.VMEM((ROWS, N), jnp.int32)],
    )(coorsT, coors)


_NC, _NS = 2, 16
_NW = _NC * _NS
_E_TOTAL = B * KNN * N
_E_PER_W = _E_TOTAL // _NW
_CHUNK = 128


def _sc_gather_kernel(idx_hbm, table_hbm, out_hbm, idx_v, buf0, buf1, sem0, sem1):
    wid = lax.axis_index("s") * _NC + lax.axis_index("c")
    base = wid * _E_PER_W
    nch = _E_PER_W // _CHUNK
    pltpu.sync_copy(idx_hbm.at[pl.ds(base, _E_PER_W)], idx_v)
    bufs = (buf0, buf1)
    sems = (sem0, sem1)
    cps = []
    for ch in range(nch):
        cp = pltpu.async_copy(table_hbm.at[idx_v.at[pl.ds(ch * _CHUNK, _CHUNK)]],
                              bufs[ch % 2], sems[ch % 2])
        cps.append(cp)
        if ch >= 1:
            cps[ch - 1].wait()
            pltpu.sync_copy(bufs[(ch - 1) % 2],
                            out_hbm.at[pl.ds(base + (ch - 1) * _CHUNK, _CHUNK)])
    cps[nch - 1].wait()
    pltpu.sync_copy(bufs[(nch - 1) % 2],
                    out_hbm.at[pl.ds(base + (nch - 1) * _CHUNK, _CHUNK)])


def _sc_gather(table, eidx):
    mesh = plsc.VectorSubcoreMesh(core_axis_name="c", subcore_axis_name="s")
    k = functools.partial(
        pl.kernel,
        mesh=mesh,
        out_type=jax.ShapeDtypeStruct((_E_TOTAL, GW), jnp.float32),
        scratch_types=[
            pltpu.VMEM((_E_PER_W,), jnp.int32),
            pltpu.VMEM((_CHUNK, GW), jnp.float32),
            pltpu.VMEM((_CHUNK, GW), jnp.float32),
            pltpu.SemaphoreType.DMA,
            pltpu.SemaphoreType.DMA,
        ],
    )(_sc_gather_kernel)
    return k(eidx, table)


def _edge_body(
    g_ref, dist_ref, feats_blk_ref, coors_blk_ref,
    w1a_ref, w1b_ref, w1d_ref, b1_ref, w2_ref, b2_ref,
    cw1_ref, cb1_ref, cw2_ref, cb2_ref, cns_ref,
    nng_ref, nnb_ref, nw1a_ref, nw1b_ref, nb1_ref, nw2_ref, nb2_ref,
    feats_out_ref, coors_out_ref,
):
    feats_blk = feats_blk_ref[0]
    coors_blk = coors_blk_ref[0]
    ne = KNN * EROWS

    gi = _dot(feats_blk, w1a_ref[...]) + b1_ref[...]

    fj = g_ref[0][:, :, :DIM].reshape(ne, DIM).astype(jnp.bfloat16)
    mval = dist_ref[0].reshape(ne, 1)
    pre = (_dot(fj, w1b_ref[...]) + mval * w1d_ref[...]).reshape(
        KNN, EROWS, 2 * EDGE_IN) + gi[None]
    h = _silu(pre).reshape(ne, 2 * EDGE_IN)
    mk = _silu(_dot(h.astype(jnp.bfloat16), w2_ref[...]) + b2_ref[...])
    cwh = _silu(_dot(mk, cw1_ref[...]) + cb1_ref[...])
    cw = _dot(cwh, cw2_ref[...]) + cb2_ref[...]
    scale = cw / jnp.clip(jnp.sqrt(mval), 1e-8) * cns_ref[0, 0]

    m_i = jnp.sum(mk.reshape(KNN, EROWS, M_DIM), axis=0)
    scale3 = scale.reshape(KNN, EROWS, 1)
    cds = []
    for c in range(3):
        xj_c = g_ref[0][:, :, DIM + c:DIM + c + 1]
        rel_c = coors_blk[None, :, c:c + 1] - xj_c
        cds.append(jnp.sum(scale3[..., 0] * rel_c[..., 0], axis=0)[:, None])
    cdelta = jnp.concatenate(cds, axis=1)

    nf = _ln(feats_blk, nng_ref[...], nnb_ref[...])
    pre2 = _dot(nf, nw1a_ref[...]) + _dot(m_i, nw1b_ref[...]) + nb1_ref[...]
    node_out = _dot(_silu(pre2), nw2_ref[...]) + nb2_ref[...] + feats_blk

    feats_out_ref[0] = node_out
    coors_out_ref[0] = coors_blk + cdelta


def _layer(feats, coors, lp):
    nbr_idx, dist_t = _topk(coors)
    boff = (jnp.arange(B, dtype=jnp.int32) * N)[:, None, None, None]
    eidx = (nbr_idx + boff).reshape(-1)
    table = jnp.concatenate(
        [feats, coors, jnp.zeros((B, N, GW - DIM - 3), jnp.float32)],
        axis=-1).reshape(B * N, GW)
    g = _sc_gather(table, eidx).reshape(B, KNN, N, GW)

    w1a = lp['e_W1'][:DIM]
    w1b = lp['e_W1'][DIM:2 * DIM].astype(jnp.bfloat16)
    w2b = lp['e_W2'].astype(jnp.bfloat16)
    w1d = lp['e_W1'][2 * DIM:2 * DIM + 1]
    nw1a = lp['n_W1'][:DIM]
    nw1b = lp['n_W1'][DIM:]
    row2 = lambda v: v.reshape(1, -1)

    grid = (B, N // EROWS)
    full = lambda shape: pl.BlockSpec(shape, lambda b, r: (0,) * len(shape))
    blk = lambda shape: pl.BlockSpec((1, EROWS) + shape, lambda b, r: (b, r) + (0,) * len(shape))
    gblk = pl.BlockSpec((1, KNN, EROWS, GW), lambda b, r: (b, 0, r, 0))
    dblk = pl.BlockSpec((1, KNN, EROWS, 1), lambda b, r: (b, 0, r, 0))

    out = pl.pallas_call(
        _edge_body,
        grid=grid,
        in_specs=[
            gblk, dblk,
            blk((DIM,)),
            blk((3,)),
            full(w1a.shape), full(w1b.shape), full((1, 2 * EDGE_IN)),
            full((1, 2 * EDGE_IN)),
            full(w2b.shape), full((1, M_DIM)),
            full(lp['c_W1'].shape), full((1, 4 * M_DIM)),
            full(lp['c_W2'].shape), full((1, 1)), full((1, 1)),
            full((1, DIM)), full((1, DIM)),
            full(nw1a.shape), full(nw1b.shape), full((1, 2 * DIM)),
            full(lp['n_W2'].shape), full((1, DIM)),
        ],
        out_specs=[blk((DIM,)), blk((3,))],
        out_shape=[
            jax.ShapeDtypeStruct((B, N, DIM), jnp.float32),
            jax.ShapeDtypeStruct((B, N, 3), jnp.float32),
        ],
    )(
        g, dist_t, feats, coors,
        w1a, w1b, row2(w1d), row2(lp['e_b1']), w2b, row2(lp['e_b2']),
        lp['c_W1'], row2(lp['c_b1']), lp['c_W2'], row2(lp['c_b2']),
        row2(lp['cn_scale']),
        row2(lp['nn_g']), row2(lp['nn_b']),
        nw1a, nw1b, row2(lp['n_b1']), lp['n_W2'], row2(lp['n_b2']),
    )
    return out[0], out[1]


def _head_body(x_ref, w1_ref, b1_ref, w2_ref, b2_ref, pa_ref,
               g1_ref, bb1_ref, g2_ref, bb2_ref, out_ref):
    x = x_ref[0]
    h = _ln(x + x, g1_ref[...], bb1_ref[...])
    a = _dot(h, w1_ref[...]) + b1_ref[...]
    a = jnp.where(a >= 0, a, pa_ref[0, 0] * a)
    h2 = _dot(a, w2_ref[...]) + b2_ref[...]
    out_ref[0] = _ln(h + h2, g2_ref[...], bb2_ref[...])


def _head(feats, params):
    row2 = lambda v: v.reshape(1, -1)
    full = lambda shape: pl.BlockSpec(shape, lambda b: (0,) * len(shape))
    return pl.pallas_call(
        _head_body,
        grid=(B,),
        in_specs=[
            pl.BlockSpec((1, N, DIM), lambda b: (b, 0, 0)),
            full(params['ffn_W1'].shape), full((1, HIDDEN * DIM)),
            full(params['ffn_W2'].shape), full((1, DIM)),
            full((1, 1)),
            full((1, DIM)), full((1, DIM)), full((1, DIM)), full((1, DIM)),
        ],
        out_specs=pl.BlockSpec((1, N, DIM), lambda b: (b, 0, 0)),
        out_shape=jax.ShapeDtypeStruct((B, N, DIM), jnp.float32),
    )(
        feats,
        params['ffn_W1'], row2(params['ffn_b1']),
        params['ffn_W2'], row2(params['ffn_b2']),
        row2(params['prelu_a']),
        row2(params['norm1_g']), row2(params['norm1_b']),
        row2(params['norm2_g']), row2(params['norm2_b']),
    )


@jax.jit
def kernel(x, z, params):
    feats = _embed(z, params['token_emb'], params['pos_emb'][:N])
    coors = x
    for lp in params['layers']:
        feats, coors = _layer(feats, coors, lp)
    h = _head(feats, params)
    return h, coors


# speedup vs baseline: 1.8425x; 1.0437x over previous
import functools

import jax
import jax.numpy as jnp
from jax import lax
from jax.experimental import pallas as pl
from jax.experimental.pallas import tpu as pltpu
from jax.experimental.pallas import tpu_sc as plsc

B, N = 2, 1024
DIM, HIDDEN = 128, 4
KNN, M_DIM = 32, 16
EDGE_IN = 2 * DIM + 1
ROWS = 512
EROWS = 128
GW = 2 * DIM


def _silu(t):
    return t * (1.0 / (1.0 + jnp.exp(-t)))


def _ln(x, g, b):
    mu = jnp.mean(x, axis=-1, keepdims=True)
    var = jnp.mean((x - mu) ** 2, axis=-1, keepdims=True)
    return (x - mu) / jnp.sqrt(var + 1e-5) * g + b


def _dot(a, b):
    return jnp.dot(a, b, preferred_element_type=jnp.float32)


def _embed_body(z_ref, tok_ref, pos_ref, out_ref):
    z = z_ref[0]
    num_tok = tok_ref.shape[0]
    cols = lax.broadcasted_iota(jnp.int32, (N, num_tok), 1)
    onehot = (z == cols).astype(jnp.float32)
    emb = jnp.dot(onehot, tok_ref[...], precision=lax.Precision.HIGHEST,
                  preferred_element_type=jnp.float32)
    out_ref[0] = emb + pos_ref[...]


def _embed(z, token_emb, pos_emb):
    z2 = z.reshape(B, N, 1).astype(jnp.int32)
    return pl.pallas_call(
        _embed_body,
        grid=(B,),
        in_specs=[
            pl.BlockSpec((1, N, 1), lambda b: (b, 0, 0)),
            pl.BlockSpec(token_emb.shape, lambda b: (0, 0)),
            pl.BlockSpec(pos_emb.shape, lambda b: (0, 0)),
        ],
        out_specs=pl.BlockSpec((1, N, DIM), lambda b: (b, 0, 0)),
        out_shape=jax.ShapeDtypeStruct((B, N, DIM), jnp.float32),
    )(z2, token_emb, pos_emb)


def _topk_body(coorsT_ref, coors_blk_ref, idx_ref, dist_ref, d_ref):
    coors_blk = coors_blk_ref[0]

    d = None
    for c in range(3):
        xi_c = coors_blk[:, c:c + 1]
        xj_c = coorsT_ref[0, c:c + 1, :]
        rel_c = xi_c - xj_c
        sq = rel_c * rel_c
        d = sq if d is None else d + sq

    cols = lax.broadcasted_iota(jnp.int32, (ROWS, N), 1)
    d_ref[...] = (lax.bitcast_convert_type(d, jnp.int32) & ~(N - 1)) | cols

    def body(k, carry):
        dcur = d_ref[...]
        mkey = jnp.min(dcur, axis=1, keepdims=True)
        hot = dcur == mkey
        d_ref[...] = jnp.where(hot, jnp.iinfo(jnp.int32).max, dcur)
        idx_ref[0, k] = mkey & (N - 1)
        dist_ref[0, k] = lax.bitcast_convert_type(mkey & ~(N - 1), jnp.float32)
        return carry

    lax.fori_loop(0, KNN, body, 0)


def _topk(coors):
    coorsT = jnp.swapaxes(coors, 1, 2)
    perb = lambda shape: pl.BlockSpec((1,) + shape, lambda b, r: (b,) + (0,) * len(shape))
    blk = lambda shape: pl.BlockSpec((1, ROWS) + shape, lambda b, r: (b, r) + (0,) * len(shape))
    kblk = lambda w: pl.BlockSpec((1, KNN, ROWS, w), lambda b, r: (b, 0, r, 0))
    return pl.pallas_call(
        _topk_body,
        grid=(B, N // ROWS),
        in_specs=[perb((3, N)), blk((3,))],
        out_specs=[kblk(1), kblk(1)],
        out_shape=[
            jax.ShapeDtypeStruct((B, KNN, N, 1), jnp.int32),
            jax.ShapeDtypeStruct((B, KNN, N, 1), jnp.float32),
        ],
        scratch_shapes=[pltpu.VMEM((ROWS, N), jnp.int32)],
    )(coorsT, coors)


_NC, _NS = 2, 16
_NW = _NC * _NS
_E_TOTAL = B * KNN * N
_E_PER_W = _E_TOTAL // _NW
_CHUNK = 128


def _sc_gather_kernel(idx_hbm, table_hbm, out_hbm, idx_v, buf0, buf1, sem0, sem1):
    wid = lax.axis_index("s") * _NC + lax.axis_index("c")
    base = wid * _E_PER_W
    nch = _E_PER_W // _CHUNK
    pltpu.sync_copy(idx_hbm.at[pl.ds(base, _E_PER_W)], idx_v)
    bufs = (buf0, buf1)
    sems = (sem0, sem1)
    cps = []
    for ch in range(nch):
        cp = pltpu.async_copy(table_hbm.at[idx_v.at[pl.ds(ch * _CHUNK, _CHUNK)]],
                              bufs[ch % 2], sems[ch % 2])
        cps.append(cp)
        if ch >= 1:
            cps[ch - 1].wait()
            pltpu.sync_copy(bufs[(ch - 1) % 2],
                            out_hbm.at[pl.ds(base + (ch - 1) * _CHUNK, _CHUNK)])
    cps[nch - 1].wait()
    pltpu.sync_copy(bufs[(nch - 1) % 2],
                    out_hbm.at[pl.ds(base + (nch - 1) * _CHUNK, _CHUNK)])


def _sc_gather(table, eidx):
    mesh = plsc.VectorSubcoreMesh(core_axis_name="c", subcore_axis_name="s")
    k = functools.partial(
        pl.kernel,
        mesh=mesh,
        out_type=jax.ShapeDtypeStruct((_E_TOTAL, GW), jnp.float32),
        scratch_types=[
            pltpu.VMEM((_E_PER_W,), jnp.int32),
            pltpu.VMEM((_CHUNK, GW), jnp.float32),
            pltpu.VMEM((_CHUNK, GW), jnp.float32),
            pltpu.SemaphoreType.DMA,
            pltpu.SemaphoreType.DMA,
        ],
    )(_sc_gather_kernel)
    return k(eidx, table)


def _edge_body(
    g_ref, dist_ref, feats_blk_ref, coors_blk_ref,
    w1a_ref, w1b_ref, w1d_ref, b1_ref, w2_ref, b2_ref,
    cw1_ref, cb1_ref, cw2_ref, cb2_ref, cns_ref,
    nng_ref, nnb_ref, nw1a_ref, nw1b_ref, nb1_ref, nw2_ref, nb2_ref,
    feats_out_ref, coors_out_ref,
):
    feats_blk = feats_blk_ref[0]
    coors_blk = coors_blk_ref[0]
    ne = KNN * EROWS

    gi = _dot(feats_blk, w1a_ref[...]) + b1_ref[...]

    fj = g_ref[0][:, :, :DIM].reshape(ne, DIM).astype(jnp.bfloat16)
    mval = dist_ref[0].reshape(ne, 1)
    pre = (_dot(fj, w1b_ref[...]) + mval * w1d_ref[...]).reshape(
        KNN, EROWS, 2 * EDGE_IN) + gi[None]
    h = _silu(pre).reshape(ne, 2 * EDGE_IN)
    mk = _silu(_dot(h.astype(jnp.bfloat16), w2_ref[...]) + b2_ref[...])
    cwh = _silu(_dot(mk, cw1_ref[...]) + cb1_ref[...])
    cw = _dot(cwh, cw2_ref[...]) + cb2_ref[...]
    scale = cw / jnp.clip(jnp.sqrt(mval), 1e-8) * cns_ref[0, 0]

    m_i = jnp.sum(mk.reshape(KNN, EROWS, M_DIM), axis=0)
    scale3 = scale.reshape(KNN, EROWS, 1)
    cds = []
    for c in range(3):
        xj_c = g_ref[0][:, :, DIM + c:DIM + c + 1]
        rel_c = coors_blk[None, :, c:c + 1] - xj_c
        cds.append(jnp.sum(scale3[..., 0] * rel_c[..., 0], axis=0)[:, None])
    cdelta = jnp.concatenate(cds, axis=1)

    nf = _ln(feats_blk, nng_ref[...], nnb_ref[...])
    pre2 = _dot(nf, nw1a_ref[...]) + _dot(m_i, nw1b_ref[...]) + nb1_ref[...]
    node_out = _dot(_silu(pre2), nw2_ref[...]) + nb2_ref[...] + feats_blk

    feats_out_ref[0] = node_out
    coors_out_ref[0] = coors_blk + cdelta


def _layer(feats, coors, lp):
    nbr_idx, dist_t = _topk(coors)
    boff = (jnp.arange(B, dtype=jnp.int32) * N)[:, None, None, None]
    eidx = (nbr_idx + boff).reshape(-1)
    table = jnp.concatenate(
        [feats, coors, jnp.zeros((B, N, GW - DIM - 3), jnp.float32)],
        axis=-1).reshape(B * N, GW)
    g = _sc_gather(table, eidx).reshape(B, KNN, N, GW)

    w1a = lp['e_W1'][:DIM]
    w1b = lp['e_W1'][DIM:2 * DIM].astype(jnp.bfloat16)
    w2b = lp['e_W2'].astype(jnp.bfloat16)
    w1d = lp['e_W1'][2 * DIM:2 * DIM + 1]
    nw1a = lp['n_W1'][:DIM]
    nw1b = lp['n_W1'][DIM:]
    row2 = lambda v: v.reshape(1, -1)

    grid = (B, N // EROWS)
    full = lambda shape: pl.BlockSpec(shape, lambda b, r: (0,) * len(shape))
    blk = lambda shape: pl.BlockSpec((1, EROWS) + shape, lambda b, r: (b, r) + (0,) * len(shape))
    gblk = pl.BlockSpec((1, KNN, EROWS, GW), lambda b, r: (b, 0, r, 0))
    dblk = pl.BlockSpec((1, KNN, EROWS, 1), lambda b, r: (b, 0, r, 0))

    out = pl.pallas_call(
        _edge_body,
        grid=grid,
        in_specs=[
            gblk, dblk,
            blk((DIM,)),
            blk((3,)),
            full(w1a.shape), full(w1b.shape), full((1, 2 * EDGE_IN)),
            full((1, 2 * EDGE_IN)),
            full(w2b.shape), full((1, M_DIM)),
            full(lp['c_W1'].shape), full((1, 4 * M_DIM)),
            full(lp['c_W2'].shape), full((1, 1)), full((1, 1)),
            full((1, DIM)), full((1, DIM)),
            full(nw1a.shape), full(nw1b.shape), full((1, 2 * DIM)),
            full(lp['n_W2'].shape), full((1, DIM)),
        ],
        out_specs=[blk((DIM,)), blk((3,))],
        out_shape=[
            jax.ShapeDtypeStruct((B, N, DIM), jnp.float32),
            jax.ShapeDtypeStruct((B, N, 3), jnp.float32),
        ],
    )(
        g, dist_t, feats, coors,
        w1a, w1b, row2(w1d), row2(lp['e_b1']), w2b, row2(lp['e_b2']),
        lp['c_W1'], row2(lp['c_b1']), lp['c_W2'], row2(lp['c_b2']),
        row2(lp['cn_scale']),
        row2(lp['nn_g']), row2(lp['nn_b']),
        nw1a, nw1b, row2(lp['n_b1']), lp['n_W2'], row2(lp['n_b2']),
    )
    return out[0], out[1]


def _head_body(x_ref, w1_ref, b1_ref, w2_ref, b2_ref, pa_ref,
               g1_ref, bb1_ref, g2_ref, bb2_ref, out_ref):
    x = x_ref[0]
    h = _ln(x + x, g1_ref[...], bb1_ref[...])
    a = _dot(h, w1_ref[...]) + b1_ref[...]
    a = jnp.where(a >= 0, a, pa_ref[0, 0] * a)
    h2 = _dot(a, w2_ref[...]) + b2_ref[...]
    out_ref[0] = _ln(h + h2, g2_ref[...], bb2_ref[...])


def _head(feats, params):
    row2 = lambda v: v.reshape(1, -1)
    full = lambda shape: pl.BlockSpec(shape, lambda b: (0,) * len(shape))
    return pl.pallas_call(
        _head_body,
        grid=(B,),
        in_specs=[
            pl.BlockSpec((1, N, DIM), lambda b: (b, 0, 0)),
            full(params['ffn_W1'].shape), full((1, HIDDEN * DIM)),
            full(params['ffn_W2'].shape), full((1, DIM)),
            full((1, 1)),
            full((1, DIM)), full((1, DIM)), full((1, DIM)), full((1, DIM)),
        ],
        out_specs=pl.BlockSpec((1, N, DIM), lambda b: (b, 0, 0)),
        out_shape=jax.ShapeDtypeStruct((B, N, DIM), jnp.float32),
    )(
        feats,
        params['ffn_W1'], row2(params['ffn_b1']),
        params['ffn_W2'], row2(params['ffn_b2']),
        row2(params['prelu_a']),
        row2(params['norm1_g']), row2(params['norm1_b']),
        row2(params['norm2_g']), row2(params['norm2_b']),
    )


@jax.jit
def kernel(x, z, params):
    feats = _embed(z, params['token_emb'], params['pos_emb'][:N])
    coors = x
    for lp in params['layers']:
        feats, coors = _layer(feats, coors, lp)
    h = _head(feats, params)
    return h, coors
